# baseline (speedup 1.0000x reference)
import functools
import math

import jax
import jax.numpy as jnp
from jax import lax
from jax.experimental import pallas as pl
from jax.experimental.pallas import tpu as pltpu
from jax.experimental.pallas import tpu_sc as plsc

D_MODEL = 64
SCALE = math.sqrt(D_MODEL)

NUM_CORES = 2
NUM_SUBCORES = 16
NUM_WORKERS = NUM_CORES * NUM_SUBCORES

CHUNK = 128


def _make_embed(total_rows: int):
    assert total_rows % (NUM_WORKERS * CHUNK) == 0
    rows_per_worker = total_rows // NUM_WORKERS
    n_chunks = rows_per_worker // CHUNK

    mesh = plsc.VectorSubcoreMesh(
        core_axis_name="c", subcore_axis_name="s"
    )

    @functools.partial(
        pl.kernel,
        out_type=jax.ShapeDtypeStruct((total_rows, D_MODEL), jnp.float32),
        mesh=mesh,
        scratch_types=[
            pltpu.VMEM((CHUNK,), jnp.int32),
            pltpu.VMEM((CHUNK, D_MODEL), jnp.float32),
            pltpu.SemaphoreType.DMA,
        ],
        compiler_params=pltpu.CompilerParams(use_tc_tiling_on_sc=False),
    )
    def embed(table_hbm, idx_hbm, out_hbm, idx_v, rows_v, sem):
        wid = lax.axis_index("s") * NUM_CORES + lax.axis_index("c")
        base = wid * rows_per_worker

        def chunk_body(g, carry):
            off = base + g * CHUNK
            pltpu.sync_copy(idx_hbm.at[pl.ds(off, CHUNK)], idx_v)
            pltpu.async_copy(table_hbm.at[idx_v], rows_v, sem).wait()

            def scale_row(i, c):
                for j in range(D_MODEL // 16):
                    s = pl.ds(j * 16, 16)
                    rows_v[i, s] = rows_v[i, s] * SCALE
                return c

            lax.fori_loop(0, CHUNK, scale_row, 0)
            pltpu.sync_copy(rows_v, out_hbm.at[pl.ds(off, CHUNK)])
            return carry

        lax.fori_loop(0, n_chunks, chunk_body, 0)

    return embed


def kernel(x, table):
    batch, seq = x.shape
    total = batch * seq
    idx = x.reshape(total).astype(jnp.int32)
    out = _make_embed(total)(table, idx)
    return out.reshape(batch, seq, D_MODEL)


# speedup vs baseline: 1.0051x; 1.0051x over previous
import functools
import math

import jax
import jax.numpy as jnp
from jax import lax
from jax.experimental import pallas as pl
from jax.experimental.pallas import tpu as pltpu
from jax.experimental.pallas import tpu_sc as plsc

D_MODEL = 64
SCALE = math.sqrt(D_MODEL)

NUM_CORES = 2
NUM_SUBCORES = 16
NUM_WORKERS = NUM_CORES * NUM_SUBCORES


def _make_embed(batch: int, seq: int):
    assert batch % NUM_WORKERS == 0
    rows_per_worker = batch // NUM_WORKERS
    subs = []
    off = 0
    while off < seq:
        n = min(128, seq - off)
        assert n % 8 == 0 and off % 8 == 0
        subs.append((off, n))
        off += n

    mesh = plsc.VectorSubcoreMesh(
        core_axis_name="c", subcore_axis_name="s"
    )

    @functools.partial(
        pl.kernel,
        out_type=jax.ShapeDtypeStruct((batch, seq, D_MODEL), jnp.float32),
        mesh=mesh,
        scratch_types=[
            pltpu.VMEM((seq,), jnp.int32),
            pltpu.VMEM((seq, D_MODEL), jnp.float32),
            pltpu.SemaphoreType.DMA,
        ],
        compiler_params=pltpu.CompilerParams(use_tc_tiling_on_sc=False),
    )
    def embed(table_hbm, idx_hbm, out_hbm, idx_v, rows_v, sem):
        wid = lax.axis_index("s") * NUM_CORES + lax.axis_index("c")
        base = wid * rows_per_worker

        def row_body(b, carry):
            pltpu.sync_copy(idx_hbm.at[base + b], idx_v)
            for (o, n) in subs:
                pltpu.async_copy(
                    table_hbm.at[idx_v.at[pl.ds(o, n)]],
                    rows_v.at[pl.ds(o, n)],
                    sem,
                ).wait()

            def scale_row(i, c):
                for j in range(D_MODEL // 16):
                    s = pl.ds(j * 16, 16)
                    rows_v[i, s] = rows_v[i, s] * SCALE
                return c

            lax.fori_loop(0, seq, scale_row, 0)
            pltpu.sync_copy(rows_v, out_hbm.at[base + b])
            return carry

        lax.fori_loop(0, rows_per_worker, row_body, 0)

    return embed


def kernel(x, table):
    batch, seq = x.shape
    out = _make_embed(batch, seq)(table, x.astype(jnp.int32))
    return out


# speedup vs baseline: 1.2570x; 1.2506x over previous
import functools
import math

import jax
import jax.numpy as jnp
from jax import lax
from jax.experimental import pallas as pl
from jax.experimental.pallas import tpu as pltpu
from jax.experimental.pallas import tpu_sc as plsc

D_MODEL = 64
SCALE = math.sqrt(D_MODEL)

NUM_CORES = 2
NUM_SUBCORES = 16
NUM_WORKERS = NUM_CORES * NUM_SUBCORES


def _make_embed(batch: int, seq: int):
    assert batch % NUM_WORKERS == 0
    n = batch // NUM_WORKERS
    assert n % 2 == 0 and seq % 8 == 0

    subs = []
    off = 0
    while off < seq:
        ln = min(128, seq - off)
        subs.append((off, ln))
        off += ln

    mesh = plsc.VectorSubcoreMesh(
        core_axis_name="c", subcore_axis_name="s"
    )

    @functools.partial(
        pl.kernel,
        out_type=jax.ShapeDtypeStruct((batch, seq, D_MODEL), jnp.float32),
        mesh=mesh,
        scratch_types=[
            pltpu.VMEM((n, seq), jnp.int32),
            pltpu.VMEM((seq, D_MODEL), jnp.float32),
            pltpu.VMEM((seq, D_MODEL), jnp.float32),
            pltpu.SemaphoreType.DMA,
            pltpu.SemaphoreType.DMA,
            pltpu.SemaphoreType.DMA,
            pltpu.SemaphoreType.DMA,
        ],
        compiler_params=pltpu.CompilerParams(use_tc_tiling_on_sc=False),
    )
    def embed(table_hbm, x_hbm, out_hbm, idx_all, rows0, rows1,
              gsem0, gsem1, ssem0, ssem1):
        wid = lax.axis_index("s") * NUM_CORES + lax.axis_index("c")
        base = wid * n
        rows = (rows0, rows1)
        gsems = (gsem0, gsem1)
        ssems = (ssem0, ssem1)

        pltpu.sync_copy(x_hbm.at[pl.ds(base, n)], idx_all)

        def start_gather(g, b):
            for (o, ln) in subs:
                pltpu.async_copy(
                    table_hbm.at[idx_all.at[g, pl.ds(o, ln)]],
                    rows[b].at[pl.ds(o, ln)],
                    gsems[b],
                )

        def wait_gather(b):
            for (o, ln) in subs:
                pltpu.make_async_copy(
                    table_hbm.at[idx_all.at[0, pl.ds(o, ln)]],
                    rows[b].at[pl.ds(o, ln)],
                    gsems[b],
                ).wait()

        def start_scatter(g, b):
            pltpu.async_copy(rows[b], out_hbm.at[base + g], ssems[b])

        def wait_scatter(b):
            pltpu.make_async_copy(rows[b], out_hbm.at[base], ssems[b]).wait()

        def scale(b):
            r = rows[b]

            @plsc.parallel_loop(0, seq, 1, unroll=4)
            def _(i):
                for j in range(D_MODEL // 16):
                    s = pl.ds(j * 16, 16)
                    r[i, s] = r[i, s] * SCALE

        start_gather(0, 0)

        def pair_body(p, carry):
            for b in range(2):
                g = 2 * p + b
                nb = 1 - b
                if b == 0:
                    @pl.when(g > 0)
                    def _():
                        wait_scatter(nb)
                    start_gather(g + 1, nb)
                else:
                    @pl.when(g < n - 1)
                    def _():
                        wait_scatter(nb)
                        start_gather(g + 1, nb)
                wait_gather(b)
                scale(b)
                start_scatter(g, b)
            return carry

        lax.fori_loop(0, n // 2, pair_body, 0)
        wait_scatter(0)
        wait_scatter(1)

    return embed


def kernel(x, table):
    batch, seq = x.shape
    return _make_embed(batch, seq)(table, x.astype(jnp.int32))
